# Initial kernel scaffold; baseline (speedup 1.0000x reference)
import jax
import jax.numpy as jnp
from jax.experimental import pallas as pl


def kernel(h, edge_index, edge_feat, W1m, b1m, W2m, b2m, W1u, b1u, W2u, b2u, gamma, beta):
    raise NotImplementedError("write your pallas kernel here")


# speedup vs baseline: 1.7123x; 1.7123x over previous
import functools

import jax
import jax.numpy as jnp
from jax import lax
from jax.experimental import pallas as pl
from jax.experimental.pallas import tpu as pltpu
from jax.experimental.pallas import tpu_sc as plsc


def _pq_body(h_ref, a_ref, b_ref, b1_ref, p_ref, q_ref):
    hb = h_ref[...]
    p_ref[...] = jnp.dot(hb, a_ref[...], preferred_element_type=jnp.float32)
    q_ref[...] = (jnp.dot(hb, b_ref[...], preferred_element_type=jnp.float32)
                  + b1_ref[0])


def _r_body(ef_ref, c_ref, r_ref):
    r_ref[...] = jnp.dot(ef_ref[...], c_ref[...],
                         preferred_element_type=jnp.float32)


_CB = 80
_D_HALF = 128
_NSUB = 16


def _zero_acc(acc, zrows, s, rows_per_sub, extra):
    rowbase = s * rows_per_sub
    nfull = rows_per_sub // _CB
    rem = rows_per_sub - nfull * _CB

    def zchunk(j, carry):
        pltpu.sync_copy(zrows, acc.at[pl.ds(rowbase + j * _CB, _CB)])
        return carry

    lax.fori_loop(0, nfull, zchunk, 0)
    if rem:
        pltpu.sync_copy(zrows.at[pl.ds(0, rem)],
                        acc.at[pl.ds(rowbase + nfull * _CB, rem)])
    if extra:
        @pl.when(s == _NSUB - 1)
        def _zx():
            pltpu.sync_copy(zrows.at[pl.ds(0, extra)],
                            acc.at[pl.ds(_NSUB * rows_per_sub, extra)])


def _write_acc(acc, out_hbm, bounce, s, out_off, rows_per_sub, extra):
    rowbase = s * rows_per_sub
    nfull = rows_per_sub // _CB
    rem = rows_per_sub - nfull * _CB

    def wchunk(j, carry):
        pltpu.sync_copy(acc.at[pl.ds(rowbase + j * _CB, _CB)], bounce)
        pltpu.sync_copy(bounce,
                        out_hbm.at[pl.ds(out_off + rowbase + j * _CB, _CB)])
        return carry

    lax.fori_loop(0, nfull, wchunk, 0)
    if rem:
        o = rowbase + nfull * _CB
        pltpu.sync_copy(acc.at[pl.ds(o, rem)], bounce.at[pl.ds(0, rem)])
        pltpu.sync_copy(bounce.at[pl.ds(0, rem)],
                        out_hbm.at[pl.ds(out_off + o, rem)])
    if extra:
        @pl.when(s == _NSUB - 1)
        def _wx():
            tb = _NSUB * rows_per_sub
            pltpu.sync_copy(acc.at[pl.ds(tb, extra)],
                            bounce.at[pl.ds(0, extra)])
            pltpu.sync_copy(bounce.at[pl.ds(0, extra)],
                            out_hbm.at[pl.ds(out_off + tb, extra)])


def _edge_body(n_nodes, n_edges, ptab, qtab, rtab, src_hbm, dst_hbm,
               s_out, s_sh, pbuf, qbuf, rbuf, srcbuf, dstbuf, gsidx, gdidx,
               sem_p, sem_q):
    c = lax.axis_index("c")
    s = lax.axis_index("s")
    epw = n_edges // _NSUB
    chunks = epw // _CB
    rows_per_sub = (n_nodes // _NSUB) // 8 * 8
    extra = n_nodes - rows_per_sub * _NSUB

    zeros16 = jnp.zeros((16,), jnp.float32)

    def init_row(i, carry):
        for j in range(_D_HALF // 16):
            pbuf[i, pl.ds(j * 16, 16)] = zeros16
        return carry

    lax.fori_loop(0, _CB, init_row, 0)
    _zero_acc(s_sh, pbuf, s, rows_per_sub, extra)
    plsc.subcore_barrier()

    ebase = s * epw
    c_off = c * n_nodes

    def chunk(k, carry):
        b = ebase + k * _CB
        pltpu.sync_copy(src_hbm.at[pl.ds(b, _CB)], srcbuf)
        pltpu.sync_copy(dst_hbm.at[pl.ds(b, _CB)], dstbuf)
        for j in range(_CB // 16):
            sl = pl.ds(j * 16, 16)
            gsidx[sl] = srcbuf[sl] + c_off
            gdidx[sl] = dstbuf[sl] + c_off
        cp = pltpu.async_copy(ptab.at[gsidx], pbuf, sem_p)
        cq = pltpu.async_copy(qtab.at[gdidx], qbuf, sem_q)
        pltpu.sync_copy(rtab.at[pl.ds(c * n_edges + b, _CB)], rbuf)
        cp.wait()
        cq.wait()

        def row(i, rc):
            for j in range(_D_HALF // 16):
                sl = pl.ds(j * 16, 16)
                z = pbuf[i, sl] + qbuf[i, sl] + rbuf[i, sl]
                pbuf[i, sl] = z / (1.0 + jnp.exp(-z))
            return rc

        lax.fori_loop(0, _CB, row, 0)
        pltpu.sync_copy(pbuf, s_sh.at[dstbuf], add=True)
        return carry

    lax.fori_loop(0, chunks, chunk, 0)
    plsc.subcore_barrier()
    _write_acc(s_sh, s_out, pbuf, s, c_off, rows_per_sub, extra)


def _deg_body(n_nodes, n_edges, dst_hbm, deg_out, deg_sh, pbuf, onesbuf,
              dstbuf, tdst):
    c = lax.axis_index("c")
    s = lax.axis_index("s")
    epw = n_edges // (2 * _NSUB)
    chunks = epw // _CB
    tail = epw - chunks * _CB
    rows_per_sub = (n_nodes // _NSUB) // 8 * 8
    extra = n_nodes - rows_per_sub * _NSUB

    zeros16 = jnp.zeros((16,), jnp.float32)
    ones16 = jnp.ones((16,), jnp.float32)

    def init_row(i, carry):
        for j in range(_D_HALF // 16):
            sl = pl.ds(j * 16, 16)
            pbuf[i, sl] = zeros16
            onesbuf[i, sl] = ones16
        return carry

    lax.fori_loop(0, _CB, init_row, 0)
    _zero_acc(deg_sh, pbuf, s, rows_per_sub, extra)
    plsc.subcore_barrier()

    ebase = (c * _NSUB + s) * epw

    def chunk(k, carry):
        pltpu.sync_copy(dst_hbm.at[pl.ds(ebase + k * _CB, _CB)], dstbuf)
        pltpu.sync_copy(onesbuf, deg_sh.at[dstbuf], add=True)
        return carry

    lax.fori_loop(0, chunks, chunk, 0)
    if tail:
        pltpu.sync_copy(dst_hbm.at[pl.ds(ebase + chunks * _CB, tail)], tdst)
        pltpu.sync_copy(onesbuf.at[pl.ds(0, tail)], deg_sh.at[tdst], add=True)
    plsc.subcore_barrier()
    _write_acc(deg_sh, deg_out, pbuf, s, c * n_nodes, rows_per_sub, extra)


def _update_body(s0_ref, s1_ref, d0_ref, d1_ref, h_ref, w2m_ref, b2m_ref,
                 w1u_ref, b1u_ref, w2u_ref, b2u_ref, g_ref, bt_ref, o_ref):
    s_full = jnp.concatenate([s0_ref[...], s1_ref[...]], axis=1)
    deg = d0_ref[:, 0:1] + d1_ref[:, 0:1]
    agg = (jnp.dot(s_full, w2m_ref[...], preferred_element_type=jnp.float32)
           + deg * b2m_ref[...])
    hb = h_ref[...]
    u1 = (jnp.dot(hb, w1u_ref[0:256, :], preferred_element_type=jnp.float32)
          + jnp.dot(agg, w1u_ref[256:512, :], preferred_element_type=jnp.float32)
          + b1u_ref[...])
    t = u1 * jax.nn.sigmoid(u1)
    h_new = jnp.dot(t, w2u_ref[...], preferred_element_type=jnp.float32) + b2u_ref[...]
    x = hb + h_new
    mu = jnp.mean(x, axis=1, keepdims=True)
    xc = x - mu
    var = jnp.mean(xc * xc, axis=1, keepdims=True)
    o_ref[...] = g_ref[...] * xc * lax.rsqrt(var + 1e-5) + bt_ref[...]


def kernel(h, edge_index, edge_feat, W1m, b1m, W2m, b2m, W1u, b1u, W2u, b2u,
           gamma, beta):
    Bx, N, D = h.shape
    E = edge_index.shape[1]
    h2 = h.reshape(N, D)
    A = W1m[0:D, :]
    B = W1m[D:2 * D, :]
    C = W1m[2 * D:, :]
    b1 = b1m.reshape(2, 1, _D_HALF)
    src = edge_index[0]
    dst = edge_index[1]
    mesh = plsc.VectorSubcoreMesh(core_axis_name="c", subcore_axis_name="s",
                                  num_cores=2, num_subcores=16)

    deg_pass = pl.kernel(
        functools.partial(_deg_body, N, E),
        out_type=jax.ShapeDtypeStruct((2 * N, _D_HALF), jnp.float32),
        mesh=mesh,
        scratch_types=[
            pltpu.VMEM_SHARED((N, _D_HALF), jnp.float32),
            pltpu.VMEM((_CB, _D_HALF), jnp.float32),
            pltpu.VMEM((_CB, _D_HALF), jnp.float32),
            pltpu.VMEM((_CB,), jnp.int32),
            pltpu.VMEM((40,), jnp.int32),
        ],
    )
    degtab = deg_pass(dst)

    BN = 400
    nb = N // BN
    ptab, qtab = pl.pallas_call(
        _pq_body,
        grid=(nb, 2),
        in_specs=[
            pl.BlockSpec((BN, D), lambda i, c: (i, 0)),
            pl.BlockSpec((D, _D_HALF), lambda i, c: (0, c)),
            pl.BlockSpec((D, _D_HALF), lambda i, c: (0, c)),
            pl.BlockSpec((1, 1, _D_HALF), lambda i, c: (c, 0, 0)),
        ],
        out_specs=[
            pl.BlockSpec((BN, _D_HALF), lambda i, c: (c * nb + i, 0)),
            pl.BlockSpec((BN, _D_HALF), lambda i, c: (c * nb + i, 0)),
        ],
        out_shape=[
            jax.ShapeDtypeStruct((2 * N, _D_HALF), jnp.float32),
            jax.ShapeDtypeStruct((2 * N, _D_HALF), jnp.float32),
        ],
    )(h2, A, B, b1)

    BE = 2000
    eb = E // BE
    rtab = pl.pallas_call(
        _r_body,
        grid=(eb, 2),
        in_specs=[
            pl.BlockSpec((BE, C.shape[0]), lambda i, c: (i, 0)),
            pl.BlockSpec((C.shape[0], _D_HALF), lambda i, c: (0, c)),
        ],
        out_specs=pl.BlockSpec((BE, _D_HALF), lambda i, c: (c * eb + i, 0)),
        out_shape=jax.ShapeDtypeStruct((2 * E, _D_HALF), jnp.float32),
    )(edge_feat, C)

    edge_pass = pl.kernel(
        functools.partial(_edge_body, N, E),
        out_type=jax.ShapeDtypeStruct((2 * N, _D_HALF), jnp.float32),
        mesh=mesh,
        scratch_types=[
            pltpu.VMEM_SHARED((N, _D_HALF), jnp.float32),
            pltpu.VMEM((_CB, _D_HALF), jnp.float32),
            pltpu.VMEM((_CB, _D_HALF), jnp.float32),
            pltpu.VMEM((_CB, _D_HALF), jnp.float32),
            pltpu.VMEM((_CB,), jnp.int32),
            pltpu.VMEM((_CB,), jnp.int32),
            pltpu.VMEM((_CB,), jnp.int32),
            pltpu.VMEM((_CB,), jnp.int32),
            pltpu.SemaphoreType.DMA,
            pltpu.SemaphoreType.DMA,
        ],
    )
    s_tab = edge_pass(ptab, qtab, rtab, src, dst)

    out2 = pl.pallas_call(
        _update_body,
        grid=(nb,),
        in_specs=[
            pl.BlockSpec((BN, _D_HALF), lambda i: (i, 0)),
            pl.BlockSpec((BN, _D_HALF), lambda i: (nb + i, 0)),
            pl.BlockSpec((BN, _D_HALF), lambda i: (i, 0)),
            pl.BlockSpec((BN, _D_HALF), lambda i: (nb + i, 0)),
            pl.BlockSpec((BN, D), lambda i: (i, 0)),
            pl.BlockSpec((D, D), lambda i: (0, 0)),
            pl.BlockSpec((1, D), lambda i: (0, 0)),
            pl.BlockSpec((2 * D, D), lambda i: (0, 0)),
            pl.BlockSpec((1, D), lambda i: (0, 0)),
            pl.BlockSpec((D, D), lambda i: (0, 0)),
            pl.BlockSpec((1, D), lambda i: (0, 0)),
            pl.BlockSpec((1, D), lambda i: (0, 0)),
            pl.BlockSpec((1, D), lambda i: (0, 0)),
        ],
        out_specs=pl.BlockSpec((BN, D), lambda i: (i, 0)),
        out_shape=jax.ShapeDtypeStruct((N, D), jnp.float32),
    )(s_tab, s_tab, degtab, degtab, h2, W2m, b2m.reshape(1, D), W1u,
      b1u.reshape(1, D), W2u, b2u.reshape(1, D), gamma.reshape(1, D),
      beta.reshape(1, D))

    return out2.reshape(Bx, N, D)
